# Initial kernel scaffold; baseline (speedup 1.0000x reference)
import jax
import jax.numpy as jnp
from jax.experimental import pallas as pl


def kernel(boxes, image_meta, p2, p3, p4, p5):
    raise NotImplementedError("write your pallas kernel here")


# speedup vs baseline: 14.5871x; 14.5871x over previous
import functools

import jax
import jax.numpy as jnp
from jax import lax
from jax.experimental import pallas as pl
from jax.experimental.pallas import tpu as pltpu
from jax.experimental.pallas import tpu_sc as plsc

POOL_H = 7
POOL_W = 7
NSAMP = POOL_H * POOL_W
SPAD = 56
WPAD = 64
NBOX = 1000
NPAD = 1024
NTILES = 32
PER_TILE = NPAD // NTILES
C = 256


OFF3 = 256 * 256
OFF4 = OFF3 + 128 * 128
OFF5 = OFF4 + 64 * 64


def _prelude_body(boxes_ref, meta_ref, idx_ref, wts_ref):
    b = boxes_ref[...]
    y1 = b[:, 0:1]
    x1 = b[:, 1:2]
    y2 = b[:, 2:3]
    x2 = b[:, 3:4]
    h = y2 - y1
    w = x2 - x1
    m = meta_ref[...]
    area = m[0, 4] * m[0, 5]
    rl = jnp.log(jnp.sqrt(h * w) / (224.0 / jnp.sqrt(area))) / jnp.log(2.0)
    lvl = jnp.minimum(5, jnp.maximum(2, 4 + jnp.round(rl).astype(jnp.int32)))
    side = jnp.right_shift(1024, lvl)
    off = jnp.where(lvl == 2, 0,
                    jnp.where(lvl == 3, OFF3,
                              jnp.where(lvl == 4, OFF4, OFF5)))
    sm1i = side - 1
    sm1f = sm1i.astype(jnp.float32)

    s = lax.broadcasted_iota(jnp.int32, (1, WPAD), 1)
    iy = (s // POOL_W).astype(jnp.float32)
    ix = (s % POOL_W).astype(jnp.float32)
    ys = y1 * sm1f + iy * (h * sm1f / (POOL_H - 1))
    xs = x1 * sm1f + ix * (w * sm1f / (POOL_W - 1))
    y0f = jnp.floor(ys)
    x0f = jnp.floor(xs)
    y0 = jnp.clip(y0f.astype(jnp.int32), 0, sm1i)
    y1c = jnp.clip(y0 + 1, 0, sm1i)
    x0 = jnp.clip(x0f.astype(jnp.int32), 0, sm1i)
    x1c = jnp.clip(x0 + 1, 0, sm1i)
    wy = ys - y0f
    wx = xs - x0f
    omy = 1.0 - wy
    omx = 1.0 - wx

    idx_ref[:, 0 * SPAD:1 * SPAD] = (off + y0 * side + x0)[:, :SPAD]
    idx_ref[:, 1 * SPAD:2 * SPAD] = (off + y0 * side + x1c)[:, :SPAD]
    idx_ref[:, 2 * SPAD:3 * SPAD] = (off + y1c * side + x0)[:, :SPAD]
    idx_ref[:, 3 * SPAD:4 * SPAD] = (off + y1c * side + x1c)[:, :SPAD]
    wts_ref[:, 0 * WPAD:1 * WPAD] = omy * omx
    wts_ref[:, 1 * WPAD:2 * WPAD] = omy * wx
    wts_ref[:, 2 * WPAD:3 * WPAD] = wy * omx
    wts_ref[:, 3 * WPAD:4 * WPAD] = wy * wx


def _prelude(boxesp, meta):
    return pl.pallas_call(
        _prelude_body,
        out_shape=[
            jax.ShapeDtypeStruct((NPAD, 4 * SPAD), jnp.int32),
            jax.ShapeDtypeStruct((NPAD, 4 * WPAD), jnp.float32),
        ],
    )(boxesp, meta)


_GD = lax.GatherDimensionNumbers(offset_dims=(), collapsed_slice_dims=(0,),
                                 start_index_map=(0,))


def _splat(vec, lane):
    return lax.gather(vec, jnp.full((16, 1), lane, jnp.int32), _GD,
                      slice_sizes=(1,),
                      mode=lax.GatherScatterMode.PROMISE_IN_BOUNDS)


def _sc_body(idx_hbm, wts_hbm, table, out_hbm,
             idx_v, wts_v, c00, c01, c10, c11, out_v, sem):
    wid = lax.axis_index("s") * 2 + lax.axis_index("c")
    base = wid * PER_TILE

    def box_body(i, carry):
        box = base + i

        @pl.when(box < NBOX)
        def _():
            pltpu.sync_copy(idx_hbm.at[box], idx_v)
            pltpu.sync_copy(wts_hbm.at[box], wts_v)
            cp0 = pltpu.async_copy(table.at[idx_v.at[0]], c00, sem)
            cp1 = pltpu.async_copy(table.at[idx_v.at[1]], c01, sem)
            cp2 = pltpu.async_copy(table.at[idx_v.at[2]], c10, sem)
            cp3 = pltpu.async_copy(table.at[idx_v.at[3]], c11, sem)
            cp0.wait()
            cp1.wait()
            cp2.wait()
            cp3.wait()

            for g in range(NSAMP // 16 + 1):
                w00v = wts_v[0, pl.ds(g * 16, 16)]
                w01v = wts_v[1, pl.ds(g * 16, 16)]
                w10v = wts_v[2, pl.ds(g * 16, 16)]
                w11v = wts_v[3, pl.ds(g * 16, 16)]
                for sl_i in range(min(16, NSAMP - g * 16)):
                    si = g * 16 + sl_i
                    w00 = _splat(w00v, sl_i)
                    w01 = _splat(w01v, sl_i)
                    w10 = _splat(w10v, sl_i)
                    w11 = _splat(w11v, sl_i)

                    def ch_body(co, c2, si=si, w00=w00, w01=w01,
                                w10=w10, w11=w11):
                        for k in range(4):
                            sl = pl.ds(co * 64 + k * 16, 16)
                            acc = (c00[si, sl] * w00 + c01[si, sl] * w01
                                   + c10[si, sl] * w10 + c11[si, sl] * w11)
                            out_v[si, sl] = acc
                        return c2

                    lax.fori_loop(0, C // 64, ch_body, 0)

            pltpu.sync_copy(out_v, out_hbm.at[box])

        return carry

    lax.fori_loop(0, PER_TILE, box_body, 0)


@functools.cache
def _sc_call():
    return functools.partial(
        pl.kernel,
        mesh=plsc.VectorSubcoreMesh(core_axis_name="c", subcore_axis_name="s"),
        out_type=jax.ShapeDtypeStruct((NBOX, NSAMP, C), jnp.float32),
        scratch_types=[
            pltpu.VMEM((4, SPAD), jnp.int32),
            pltpu.VMEM((4, WPAD), jnp.float32),
            pltpu.VMEM((SPAD, C), jnp.float32),
            pltpu.VMEM((SPAD, C), jnp.float32),
            pltpu.VMEM((SPAD, C), jnp.float32),
            pltpu.VMEM((SPAD, C), jnp.float32),
            pltpu.VMEM((NSAMP, C), jnp.float32),
            pltpu.SemaphoreType.DMA,
        ],
    )(_sc_body)


def kernel(boxes, image_meta, p2, p3, p4, p5):
    boxes2 = boxes.reshape(-1, 4)
    boxesp = jnp.pad(boxes2, ((0, NPAD - boxes2.shape[0]), (0, 0)))
    idx_flat, wts_flat = _prelude(boxesp, image_meta)
    idx3 = idx_flat.reshape(NPAD, 4, SPAD)
    wts3 = wts_flat.reshape(NPAD, 4, WPAD)
    table = jnp.concatenate(
        [p2.reshape(-1, C), p3.reshape(-1, C),
         p4.reshape(-1, C), p5.reshape(-1, C)], axis=0)
    out = _sc_call()(idx3, wts3, table)
    return out.reshape(1, NBOX, POOL_H, POOL_W, C)
